# Initial kernel scaffold; baseline (speedup 1.0000x reference)
import jax
import jax.numpy as jnp
from jax.experimental import pallas as pl


def kernel(x):
    raise NotImplementedError("write your pallas kernel here")


# speedup vs baseline: 2.8987x; 2.8987x over previous
import functools

import numpy as np
import jax
import jax.numpy as jnp
from jax import lax
from jax.experimental import pallas as pl
from jax.experimental.pallas import tpu as pltpu
from jax.experimental.pallas import tpu_sc as plsc

N = 8192
N1 = 64
N2 = 128
K2 = 65
F = K2 * N1
FV = N // 2 + 1
KTOP = 64
EPS = 1e-6
DBLK = 128

_HIGH = lax.Precision.HIGHEST


def _constants():
    n1 = np.arange(N1)
    k1 = np.arange(N1)
    n2 = np.arange(N2)
    k2 = np.arange(K2)
    ang1 = -2.0 * np.pi * np.outer(k1, n1) / N1
    w1r, w1i = np.cos(ang1), np.sin(ang1)
    angt = -2.0 * np.pi * np.outer(k1, n2) / N
    tr, ti = np.cos(angt), np.sin(angt)
    ange = -2.0 * np.pi * np.outer(k2, n2) / N2
    er, ei = np.cos(ange), np.sin(ange)
    b = np.arange(N2)
    a = np.arange(N1)
    ang2 = 2.0 * np.pi * np.outer(b, k2) / N2
    e2r, e2i = np.cos(ang2), np.sin(ang2)
    angw = 2.0 * np.pi * np.outer(b, k1) / N
    twr, twi = np.cos(angw), np.sin(angw)
    ang3 = 2.0 * np.pi * np.outer(a, k1) / N1
    e1r, e1i = np.cos(ang3), np.sin(ang3)
    k = (k2[:, None] * N1 + k1[None, :])
    wg = np.where(k > N // 2, 0.0, np.where((k == 0) | (k == N // 2), 1.0, 2.0)) / N
    c = lambda m: jnp.asarray(m, jnp.float32)
    return {
        "w1r": c(w1r), "w1i": c(w1i), "tr": c(tr), "ti": c(ti),
        "er": c(er), "ei": c(ei),
        "e2r": c(e2r), "e2i": c(e2i), "twr": c(twr), "twi": c(twi),
        "e1r": c(e1r), "e1i": c(e1i), "wg": c(wg),
    }


def _dot(a, b):
    return jnp.dot(a, b, preferred_element_type=jnp.float32, precision=_HIGH)


def _fwd_body(x_ref, w1r, w1i, tr, ti, er, ei, xr_out, xi_out, amp_out):
    xb = x_ref[0]
    x2 = xb.reshape(N1, N2 * DBLK)
    yr = _dot(w1r[...], x2).reshape(N1, N2, DBLK)
    yi = _dot(w1i[...], x2).reshape(N1, N2, DBLK)
    trc = tr[...][:, :, None]
    tic = ti[...][:, :, None]
    zr = yr * trc - yi * tic
    zi = yr * tic + yi * trc
    zr = jnp.swapaxes(zr, 0, 1).reshape(N2, N1 * DBLK)
    zi = jnp.swapaxes(zi, 0, 1).reshape(N2, N1 * DBLK)
    erc, eic = er[...], ei[...]
    xr = _dot(erc, zr) - _dot(eic, zi)
    xi = _dot(erc, zi) + _dot(eic, zr)
    xr = xr.reshape(F, DBLK)
    xi = xi.reshape(F, DBLK)
    amp = jnp.sqrt(jnp.square(xr + EPS) + jnp.square(xi + EPS))
    row = lax.broadcasted_iota(jnp.int32, (F, DBLK), 0)
    amp = jnp.where(row < FV, amp, 0.0)
    xr_out[0] = xr
    xi_out[0] = xi
    amp_out[0] = amp.T


def _forward(x, c, interpret=False):
    B, _, D = x.shape
    grid = (B, D // DBLK)
    spec_small = lambda s: pl.BlockSpec(s, lambda b, j: (0,) * len(s))
    out = pl.pallas_call(
        _fwd_body,
        grid=grid,
        in_specs=[
            pl.BlockSpec((1, N, DBLK), lambda b, j: (b, 0, j)),
            spec_small((N1, N1)), spec_small((N1, N1)),
            spec_small((N1, N2)), spec_small((N1, N2)),
            spec_small((K2, N2)), spec_small((K2, N2)),
        ],
        out_specs=[pl.BlockSpec((1, F, DBLK), lambda b, j: (b, 0, j))] * 2
        + [pl.BlockSpec((1, DBLK, F), lambda b, j: (b, j, 0))],
        out_shape=[jax.ShapeDtypeStruct((B, F, D), jnp.float32)] * 2
        + [jax.ShapeDtypeStruct((B, D, F), jnp.float32)],
        interpret=interpret,
    )(x, c["w1r"], c["w1i"], c["tr"], c["ti"], c["er"], c["ei"])
    return out


def _inv_body(x_ref, xr_ref, xi_ref, thr_ref,
              e2r, e2i, twr, twi, e1r, e1i, wg,
              rec_out, res_out):
    xr = xr_ref[0]
    xi = xi_ref[0]
    amp = jnp.sqrt(jnp.square(xr + EPS) + jnp.square(xi + EPS))
    thr = thr_ref[0]
    keep = (amp >= thr).astype(jnp.float32)
    wgc = wg[...].reshape(F, 1)
    yhr = (xr * keep * wgc).reshape(K2, N1 * DBLK)
    yhi = (xi * keep * wgc).reshape(K2, N1 * DBLK)
    e2rc, e2ic = e2r[...], e2i[...]
    ur = (_dot(e2rc, yhr) - _dot(e2ic, yhi)).reshape(N2, N1, DBLK)
    ui = (_dot(e2rc, yhi) + _dot(e2ic, yhr)).reshape(N2, N1, DBLK)
    twrc = twr[...][:, :, None]
    twic = twi[...][:, :, None]
    vr = ur * twrc - ui * twic
    vi = ur * twic + ui * twrc
    vr = jnp.swapaxes(vr, 0, 1).reshape(N1, N2 * DBLK)
    vi = jnp.swapaxes(vi, 0, 1).reshape(N1, N2 * DBLK)
    rec = (_dot(e1r[...], vr) - _dot(e1i[...], vi)).reshape(N, DBLK)
    rec_out[0] = rec
    res_out[0] = x_ref[0] - rec


def _inverse(x, xr, xi, thr, c, interpret=False):
    B, _, D = x.shape
    grid = (B, D // DBLK)
    spec_small = lambda s: pl.BlockSpec(s, lambda b, j: (0,) * len(s))
    rec, res = pl.pallas_call(
        _inv_body,
        grid=grid,
        in_specs=[
            pl.BlockSpec((1, N, DBLK), lambda b, j: (b, 0, j)),
            pl.BlockSpec((1, F, DBLK), lambda b, j: (b, 0, j)),
            pl.BlockSpec((1, F, DBLK), lambda b, j: (b, 0, j)),
            pl.BlockSpec((1, 1, DBLK), lambda b, j: (b, 0, j)),
            spec_small((N2, K2)), spec_small((N2, K2)),
            spec_small((N2, N1)), spec_small((N2, N1)),
            spec_small((N1, N1)), spec_small((N1, N1)),
            spec_small((F,)),
        ],
        out_specs=[pl.BlockSpec((1, N, DBLK), lambda b, j: (b, 0, j))] * 2,
        out_shape=[jax.ShapeDtypeStruct((B, N, D), jnp.float32)] * 2,
        interpret=interpret,
    )(x, xr, xi, thr.reshape(B, 1, D), c["e2r"], c["e2i"], c["twr"], c["twi"],
      c["e1r"], c["e1i"], c["wg"].reshape(F))
    return rec, res


_CTILES = 16


def _make_select(B, D):
    nblocks = (B * D) // _CTILES
    nworkers = 32
    per_w = nblocks // nworkers
    dtiles = D // _CTILES

    mesh = plsc.VectorSubcoreMesh(core_axis_name="c", subcore_axis_name="s")

    @functools.partial(
        pl.kernel,
        out_type=jax.ShapeDtypeStruct((B, D), jnp.float32),
        mesh=mesh,
        compiler_params=pltpu.CompilerParams(
            use_tc_tiling_on_sc=False, needs_layout_passes=False),
        scratch_types=[
            pltpu.VMEM((_CTILES * F,), jnp.float32),
            pltpu.VMEM((256, _CTILES), jnp.int32),
            pltpu.VMEM((_CTILES,), jnp.float32),
        ],
    )
    def select(amp_hbm, thr_hbm, amp_v, hist_v, thr_v):
        wid = lax.axis_index("s") * 2 + lax.axis_index("c")
        lanes = lax.iota(jnp.int32, 16)
        ones = jnp.full((16,), 1, jnp.int32)

        def do_block(i, _):
            g = wid * per_w + i
            b = g // dtiles
            d0 = (g % dtiles) * _CTILES
            for l in range(_CTILES):
                pltpu.sync_copy(amp_hbm.at[b, d0 + l, :],
                                amp_v.at[pl.ds(l * F, F)])
            lane_base = lanes * F

            def run_pass(p, carry):
                prefix, need, himask = carry
                shift = 24 - 8 * p

                def zero(j, _):
                    hist_v[j, :] = jnp.zeros((16,), jnp.int32)
                    return 0
                lax.fori_loop(0, 256, zero, 0)

                def count(r, _):
                    a = plsc.load_gather(amp_v, [lane_base + r])
                    u = plsc.bitcast(a, jnp.int32)
                    m = (u & himask) == prefix
                    bk = (u >> shift) & 255
                    plsc.addupdate_scatter(hist_v, [bk, lanes], ones, mask=m)
                    return 0
                lax.fori_loop(0, F, count, 0)

                def scan(jj, sc):
                    j = 255 - jj
                    acc, need, prefix, done = sc
                    h = hist_v[j, :]
                    accn = acc + h
                    selm = jnp.logical_and(jnp.logical_not(done), accn >= need)
                    prefix = jnp.where(selm, prefix | (j << shift), prefix)
                    need = jnp.where(selm, need - acc, need)
                    done = jnp.logical_or(done, selm)
                    return (accn, need, prefix, done)
                zero16 = jnp.zeros((16,), jnp.int32)
                _, need, prefix, _ = lax.fori_loop(
                    0, 256, scan,
                    (zero16, need, prefix, zero16 > 0))
                himask = himask | (255 << shift)
                return (prefix, need, himask)

            carry = (jnp.zeros((16,), jnp.int32),
                     jnp.full((16,), KTOP, jnp.int32),
                     jnp.zeros((16,), jnp.int32))
            prefix, _, _ = lax.fori_loop(0, 4, run_pass, carry)
            thr_v[...] = plsc.bitcast(prefix, jnp.float32)
            pltpu.sync_copy(thr_v, thr_hbm.at[b, pl.ds(d0, _CTILES)])
            return 0

        lax.fori_loop(0, per_w, do_block, 0)

    return select


def kernel(x):
    B, _, D = x.shape
    c = _constants()
    xr, xi, amp = _forward(x, c)
    thr = _make_select(B, D)(amp)
    rec, res = _inverse(x, xr, xi, thr, c)
    return (rec, res)


# speedup vs baseline: 3.4094x; 1.1762x over previous
import functools

import numpy as np
import jax
import jax.numpy as jnp
from jax import lax
from jax.experimental import pallas as pl
from jax.experimental.pallas import tpu as pltpu
from jax.experimental.pallas import tpu_sc as plsc

N = 8192
N1 = 64
N2 = 128
K2 = 65
F = K2 * N1
FV = N // 2 + 1
KTOP = 64
EPS = 1e-6
DBLK = 128

_HIGH = lax.Precision.HIGHEST


def _constants():
    n1 = np.arange(N1)
    k1 = np.arange(N1)
    n2 = np.arange(N2)
    k2 = np.arange(K2)
    ang1 = -2.0 * np.pi * np.outer(k1, n1) / N1
    w1r, w1i = np.cos(ang1), np.sin(ang1)
    angt = -2.0 * np.pi * np.outer(k1, n2) / N
    tr, ti = np.cos(angt), np.sin(angt)
    ange = -2.0 * np.pi * np.outer(k2, n2) / N2
    er, ei = np.cos(ange), np.sin(ange)
    b = np.arange(N2)
    a = np.arange(N1)
    ang2 = 2.0 * np.pi * np.outer(b, k2) / N2
    e2r, e2i = np.cos(ang2), np.sin(ang2)
    angw = 2.0 * np.pi * np.outer(b, k1) / N
    twr, twi = np.cos(angw), np.sin(angw)
    ang3 = 2.0 * np.pi * np.outer(a, k1) / N1
    e1r, e1i = np.cos(ang3), np.sin(ang3)
    k = (k2[:, None] * N1 + k1[None, :])
    wg = np.where(k > N // 2, 0.0, np.where((k == 0) | (k == N // 2), 1.0, 2.0)) / N
    c = lambda m: jnp.asarray(m, jnp.float32)
    return {
        "w1r": c(w1r), "w1i": c(w1i), "tr": c(tr), "ti": c(ti),
        "er": c(er), "ei": c(ei),
        "e2r": c(e2r), "e2i": c(e2i), "twr": c(twr), "twi": c(twi),
        "e1r": c(e1r), "e1i": c(e1i), "wg": c(wg),
    }


def _dot(a, b):
    return jnp.dot(a, b, preferred_element_type=jnp.float32, precision=_HIGH)


def _fwd_body(x_ref, w1r, w1i, tr, ti, er, ei, xr_out, xi_out, amp_out):
    xb = x_ref[0]
    x2 = xb.reshape(N1, N2 * DBLK)
    yr = _dot(w1r[...], x2).reshape(N1, N2, DBLK)
    yi = _dot(w1i[...], x2).reshape(N1, N2, DBLK)
    trc = tr[...][:, :, None]
    tic = ti[...][:, :, None]
    zr = yr * trc - yi * tic
    zi = yr * tic + yi * trc
    zr = jnp.swapaxes(zr, 0, 1).reshape(N2, N1 * DBLK)
    zi = jnp.swapaxes(zi, 0, 1).reshape(N2, N1 * DBLK)
    erc, eic = er[...], ei[...]
    xr = _dot(erc, zr) - _dot(eic, zi)
    xi = _dot(erc, zi) + _dot(eic, zr)
    xr = xr.reshape(F, DBLK)
    xi = xi.reshape(F, DBLK)
    amp = jnp.sqrt(jnp.square(xr + EPS) + jnp.square(xi + EPS))
    row = lax.broadcasted_iota(jnp.int32, (F, DBLK), 0)
    amp = jnp.where(row < FV, amp, 0.0)
    xr_out[0] = xr
    xi_out[0] = xi
    amp_out[0] = amp.T


def _forward(x, c, interpret=False):
    B, _, D = x.shape
    grid = (B, D // DBLK)
    spec_small = lambda s: pl.BlockSpec(s, lambda b, j: (0,) * len(s))
    out = pl.pallas_call(
        _fwd_body,
        grid=grid,
        in_specs=[
            pl.BlockSpec((1, N, DBLK), lambda b, j: (b, 0, j)),
            spec_small((N1, N1)), spec_small((N1, N1)),
            spec_small((N1, N2)), spec_small((N1, N2)),
            spec_small((K2, N2)), spec_small((K2, N2)),
        ],
        out_specs=[pl.BlockSpec((1, F, DBLK), lambda b, j: (b, 0, j))] * 2
        + [pl.BlockSpec((1, DBLK, F), lambda b, j: (b, j, 0))],
        out_shape=[jax.ShapeDtypeStruct((B, F, D), jnp.float32)] * 2
        + [jax.ShapeDtypeStruct((B, D, F), jnp.float32)],
        interpret=interpret,
    )(x, c["w1r"], c["w1i"], c["tr"], c["ti"], c["er"], c["ei"])
    return out


def _inv_body(x_ref, xr_ref, xi_ref, thr_ref,
              e2r, e2i, twr, twi, e1r, e1i, wg,
              rec_out, res_out):
    xr = xr_ref[0]
    xi = xi_ref[0]
    amp = jnp.sqrt(jnp.square(xr + EPS) + jnp.square(xi + EPS))
    thr = thr_ref[0]
    keep = (amp >= thr).astype(jnp.float32)
    wgc = wg[...].reshape(F, 1)
    yhr = (xr * keep * wgc).reshape(K2, N1 * DBLK)
    yhi = (xi * keep * wgc).reshape(K2, N1 * DBLK)
    e2rc, e2ic = e2r[...], e2i[...]
    ur = (_dot(e2rc, yhr) - _dot(e2ic, yhi)).reshape(N2, N1, DBLK)
    ui = (_dot(e2rc, yhi) + _dot(e2ic, yhr)).reshape(N2, N1, DBLK)
    twrc = twr[...][:, :, None]
    twic = twi[...][:, :, None]
    vr = ur * twrc - ui * twic
    vi = ur * twic + ui * twrc
    vr = jnp.swapaxes(vr, 0, 1).reshape(N1, N2 * DBLK)
    vi = jnp.swapaxes(vi, 0, 1).reshape(N1, N2 * DBLK)
    rec = (_dot(e1r[...], vr) - _dot(e1i[...], vi)).reshape(N, DBLK)
    rec_out[0] = rec
    res_out[0] = x_ref[0] - rec


def _inverse(x, xr, xi, thr, c, interpret=False):
    B, _, D = x.shape
    grid = (B, D // DBLK)
    spec_small = lambda s: pl.BlockSpec(s, lambda b, j: (0,) * len(s))
    rec, res = pl.pallas_call(
        _inv_body,
        grid=grid,
        in_specs=[
            pl.BlockSpec((1, N, DBLK), lambda b, j: (b, 0, j)),
            pl.BlockSpec((1, F, DBLK), lambda b, j: (b, 0, j)),
            pl.BlockSpec((1, F, DBLK), lambda b, j: (b, 0, j)),
            pl.BlockSpec((1, 1, DBLK), lambda b, j: (b, 0, j)),
            spec_small((N2, K2)), spec_small((N2, K2)),
            spec_small((N2, N1)), spec_small((N2, N1)),
            spec_small((N1, N1)), spec_small((N1, N1)),
            spec_small((F,)),
        ],
        out_specs=[pl.BlockSpec((1, N, DBLK), lambda b, j: (b, 0, j))] * 2,
        out_shape=[jax.ShapeDtypeStruct((B, N, D), jnp.float32)] * 2,
        interpret=interpret,
    )(x, xr, xi, thr.reshape(B, 1, D), c["e2r"], c["e2i"], c["twr"], c["twi"],
      c["e1r"], c["e1i"], c["wg"].reshape(F))
    return rec, res


_CTILES = 16


def _make_select(B, D):
    nblocks = (B * D) // _CTILES
    nworkers = 32
    per_w = nblocks // nworkers
    dtiles = D // _CTILES

    mesh = plsc.VectorSubcoreMesh(core_axis_name="c", subcore_axis_name="s")

    @functools.partial(
        pl.kernel,
        out_type=jax.ShapeDtypeStruct((B, D), jnp.float32),
        mesh=mesh,
        compiler_params=pltpu.CompilerParams(
            use_tc_tiling_on_sc=False, needs_layout_passes=False),
        scratch_types=[
            pltpu.VMEM((_CTILES * F,), jnp.float32),
            pltpu.VMEM((256, _CTILES), jnp.int32),
            pltpu.VMEM((_CTILES,), jnp.float32),
            pltpu.SemaphoreType.DMA,
        ],
    )
    def select(amp_hbm, thr_hbm, amp_v, hist_v, thr_v, dsem):
        wid = lax.axis_index("s") * 2 + lax.axis_index("c")
        lanes = lax.iota(jnp.int32, 16)
        ones = jnp.full((16,), 1, jnp.int32)
        zero16 = jnp.zeros((16,), jnp.int32)
        UN = 8

        def do_block(i, _):
            g = wid * per_w + i
            b = g // dtiles
            d0 = (g % dtiles) * _CTILES
            copies = [pltpu.async_copy(amp_hbm.at[b, d0 + l, :],
                                       amp_v.at[pl.ds(l * F, F)], dsem)
                      for l in range(_CTILES)]
            for cp in copies:
                cp.wait()
            lane_base = lanes * F

            prefix = zero16
            need = jnp.full((16,), KTOP, jnp.int32)
            for p in range(4):
                shift = 24 - 8 * p
                hm_py = (0xFFFFFFFF << (32 - 8 * p)) & 0xFFFFFFFF if p else 0
                himask = jnp.int32(hm_py - (1 << 32) if hm_py >= (1 << 31) else hm_py)

                def zero(r, _):
                    for j in range(UN):
                        hist_v[r * UN + j, :] = zero16
                    return 0
                lax.fori_loop(0, 256 // UN, zero, 0, unroll=False)

                def count(r, _, p=p, shift=shift, himask=himask, prefix=prefix):
                    for j in range(UN):
                        a = plsc.load_gather(amp_v, [lane_base + (r * UN + j)])
                        u = plsc.bitcast(a, jnp.int32)
                        bk = (u >> shift) & 255
                        if p == 0:
                            plsc.addupdate_scatter(hist_v, [bk, lanes], ones)
                        else:
                            m = (u & himask) == prefix
                            plsc.addupdate_scatter(hist_v, [bk, lanes], ones,
                                                   mask=m)
                    return 0
                lax.fori_loop(0, F // UN, count, 0, unroll=False)

                def scan(r, sc, shift=shift):
                    acc, need, prefix, done = sc
                    for j in range(UN):
                        jj = 255 - (r * UN + j)
                        h = hist_v[jj, :]
                        accn = acc + h
                        selm = jnp.logical_and(jnp.logical_not(done),
                                               accn >= need)
                        prefix = jnp.where(selm, prefix | (jj << shift), prefix)
                        need = jnp.where(selm, need - acc, need)
                        done = jnp.logical_or(done, selm)
                        acc = accn
                    return (acc, need, prefix, done)
                _, need, prefix, _ = lax.fori_loop(
                    0, 256 // UN, scan,
                    (zero16, need, prefix, zero16 > 0), unroll=False)

            thr_v[...] = plsc.bitcast(prefix, jnp.float32)
            pltpu.sync_copy(thr_v, thr_hbm.at[b, pl.ds(d0, _CTILES)])
            return 0

        lax.fori_loop(0, per_w, do_block, 0)

    return select


def kernel(x):
    B, _, D = x.shape
    c = _constants()
    xr, xi, amp = _forward(x, c)
    thr = _make_select(B, D)(amp)
    rec, res = _inverse(x, xr, xi, thr, c)
    return (rec, res)


# speedup vs baseline: 5.1798x; 1.5193x over previous
import functools

import numpy as np
import jax
import jax.numpy as jnp
from jax import lax
from jax.experimental import pallas as pl
from jax.experimental.pallas import tpu as pltpu
from jax.experimental.pallas import tpu_sc as plsc

N = 8192
N1 = 64
N2 = 128
K2 = 65
F = K2 * N1
FV = N // 2 + 1
KTOP = 64
EPS = 1e-6
DBLK = 128

_HIGH = lax.Precision.HIGHEST


def _constants():
    n1 = np.arange(N1)
    k1 = np.arange(N1)
    n2 = np.arange(N2)
    k2 = np.arange(K2)
    ang1 = -2.0 * np.pi * np.outer(k1, n1) / N1
    w1r, w1i = np.cos(ang1), np.sin(ang1)
    angt = -2.0 * np.pi * np.outer(k1, n2) / N
    tr, ti = np.cos(angt), np.sin(angt)
    ange = -2.0 * np.pi * np.outer(k2, n2) / N2
    er, ei = np.cos(ange), np.sin(ange)
    b = np.arange(N2)
    a = np.arange(N1)
    ang2 = 2.0 * np.pi * np.outer(b, k2) / N2
    e2r, e2i = np.cos(ang2), np.sin(ang2)
    angw = 2.0 * np.pi * np.outer(b, k1) / N
    twr, twi = np.cos(angw), np.sin(angw)
    ang3 = 2.0 * np.pi * np.outer(a, k1) / N1
    e1r, e1i = np.cos(ang3), np.sin(ang3)
    k = (k2[:, None] * N1 + k1[None, :])
    wg = np.where(k > N // 2, 0.0, np.where((k == 0) | (k == N // 2), 1.0, 2.0)) / N
    c = lambda m: jnp.asarray(m, jnp.float32)
    return {
        "w1r": c(w1r), "w1i": c(w1i), "tr": c(tr), "ti": c(ti),
        "er": c(er), "ei": c(ei),
        "e2r": c(e2r), "e2i": c(e2i), "twr": c(twr), "twi": c(twi),
        "e1r": c(e1r), "e1i": c(e1i), "wg": c(wg),
    }


def _dot(a, b):
    return jnp.dot(a, b, preferred_element_type=jnp.float32, precision=_HIGH)


def _fwd_body(x_ref, w1r, w1i, tr, ti, er, ei, xr_out, xi_out, amp_out):
    xb = x_ref[0]
    x2 = xb.reshape(N1, N2 * DBLK)
    yr = _dot(w1r[...], x2).reshape(N1, N2, DBLK)
    yi = _dot(w1i[...], x2).reshape(N1, N2, DBLK)
    trc = tr[...][:, :, None]
    tic = ti[...][:, :, None]
    zr = yr * trc - yi * tic
    zi = yr * tic + yi * trc
    zr = jnp.swapaxes(zr, 0, 1).reshape(N2, N1 * DBLK)
    zi = jnp.swapaxes(zi, 0, 1).reshape(N2, N1 * DBLK)
    erc, eic = er[...], ei[...]
    xr = _dot(erc, zr) - _dot(eic, zi)
    xi = _dot(erc, zi) + _dot(eic, zr)
    xr = xr.reshape(F, DBLK)
    xi = xi.reshape(F, DBLK)
    amp = jnp.sqrt(jnp.square(xr + EPS) + jnp.square(xi + EPS))
    row = lax.broadcasted_iota(jnp.int32, (F, DBLK), 0)
    amp = jnp.where(row < FV, amp, 0.0)
    xr_out[0] = xr
    xi_out[0] = xi
    amp_out[0] = amp.T


def _forward(x, c, interpret=False):
    B, _, D = x.shape
    grid = (B, D // DBLK)
    spec_small = lambda s: pl.BlockSpec(s, lambda b, j: (0,) * len(s))
    out = pl.pallas_call(
        _fwd_body,
        grid=grid,
        in_specs=[
            pl.BlockSpec((1, N, DBLK), lambda b, j: (b, 0, j)),
            spec_small((N1, N1)), spec_small((N1, N1)),
            spec_small((N1, N2)), spec_small((N1, N2)),
            spec_small((K2, N2)), spec_small((K2, N2)),
        ],
        out_specs=[pl.BlockSpec((1, F, DBLK), lambda b, j: (b, 0, j))] * 2
        + [pl.BlockSpec((1, DBLK, F), lambda b, j: (b, j, 0))],
        out_shape=[jax.ShapeDtypeStruct((B, F, D), jnp.float32)] * 2
        + [jax.ShapeDtypeStruct((B, D, F), jnp.float32)],
        interpret=interpret,
    )(x, c["w1r"], c["w1i"], c["tr"], c["ti"], c["er"], c["ei"])
    return out


def _inv_body(x_ref, xr_ref, xi_ref, thr_ref,
              e2r, e2i, twr, twi, e1r, e1i, wg,
              rec_out, res_out):
    xr = xr_ref[0]
    xi = xi_ref[0]
    amp = jnp.sqrt(jnp.square(xr + EPS) + jnp.square(xi + EPS))
    thr = thr_ref[0]
    keep = (amp >= thr).astype(jnp.float32)
    wgc = wg[...].reshape(F, 1)
    yhr = (xr * keep * wgc).reshape(K2, N1 * DBLK)
    yhi = (xi * keep * wgc).reshape(K2, N1 * DBLK)
    e2rc, e2ic = e2r[...], e2i[...]
    ur = (_dot(e2rc, yhr) - _dot(e2ic, yhi)).reshape(N2, N1, DBLK)
    ui = (_dot(e2rc, yhi) + _dot(e2ic, yhr)).reshape(N2, N1, DBLK)
    twrc = twr[...][:, :, None]
    twic = twi[...][:, :, None]
    vr = ur * twrc - ui * twic
    vi = ur * twic + ui * twrc
    vr = jnp.swapaxes(vr, 0, 1).reshape(N1, N2 * DBLK)
    vi = jnp.swapaxes(vi, 0, 1).reshape(N1, N2 * DBLK)
    rec = (_dot(e1r[...], vr) - _dot(e1i[...], vi)).reshape(N, DBLK)
    rec_out[0] = rec
    res_out[0] = x_ref[0] - rec


def _inverse(x, xr, xi, thr, c, interpret=False):
    B, _, D = x.shape
    grid = (B, D // DBLK)
    spec_small = lambda s: pl.BlockSpec(s, lambda b, j: (0,) * len(s))
    rec, res = pl.pallas_call(
        _inv_body,
        grid=grid,
        in_specs=[
            pl.BlockSpec((1, N, DBLK), lambda b, j: (b, 0, j)),
            pl.BlockSpec((1, F, DBLK), lambda b, j: (b, 0, j)),
            pl.BlockSpec((1, F, DBLK), lambda b, j: (b, 0, j)),
            pl.BlockSpec((1, 1, DBLK), lambda b, j: (b, 0, j)),
            spec_small((N2, K2)), spec_small((N2, K2)),
            spec_small((N2, N1)), spec_small((N2, N1)),
            spec_small((N1, N1)), spec_small((N1, N1)),
            spec_small((F,)),
        ],
        out_specs=[pl.BlockSpec((1, N, DBLK), lambda b, j: (b, 0, j))] * 2,
        out_shape=[jax.ShapeDtypeStruct((B, N, D), jnp.float32)] * 2,
        interpret=interpret,
    )(x, xr, xi, thr.reshape(B, 1, D), c["e2r"], c["e2i"], c["twr"], c["twi"],
      c["e1r"], c["e1i"], c["wg"].reshape(F))
    return rec, res


_CTILES = 16


def _make_select(B, D):
    nblocks = (B * D) // _CTILES
    nworkers = 32
    per_w = nblocks // nworkers
    dtiles = D // _CTILES

    mesh = plsc.VectorSubcoreMesh(core_axis_name="c", subcore_axis_name="s")

    @functools.partial(
        pl.kernel,
        out_type=jax.ShapeDtypeStruct((B, D), jnp.float32),
        mesh=mesh,
        compiler_params=pltpu.CompilerParams(
            use_tc_tiling_on_sc=False, needs_layout_passes=False),
        scratch_types=[
            pltpu.VMEM((_CTILES * F,), jnp.float32),
            pltpu.VMEM((256, _CTILES), jnp.int32),
            pltpu.VMEM((_CTILES,), jnp.float32),
            pltpu.SemaphoreType.DMA,
        ],
    )
    def select(amp_hbm, thr_hbm, amp_v, hist_v, thr_v, dsem):
        wid = lax.axis_index("s") * 2 + lax.axis_index("c")
        lanes = lax.iota(jnp.int32, 16)
        ones = jnp.full((16,), 1, jnp.int32)
        zero16 = jnp.zeros((16,), jnp.int32)
        UN = 8

        def do_block(i, _):
            g = wid * per_w + i
            b = g // dtiles
            d0 = (g % dtiles) * _CTILES
            copies = [pltpu.async_copy(amp_hbm.at[b, d0 + l, :],
                                       amp_v.at[pl.ds(l * F, F)], dsem)
                      for l in range(_CTILES)]
            for cp in copies:
                cp.wait()
            lane_base = lanes * F

            prefix = zero16
            need = jnp.full((16,), KTOP, jnp.int32)
            for p in range(4):
                shift = 24 - 8 * p
                hm_py = (0xFFFFFFFF << (32 - 8 * p)) & 0xFFFFFFFF if p else 0
                himask = jnp.int32(hm_py - (1 << 32) if hm_py >= (1 << 31) else hm_py)

                @plsc.parallel_loop(0, 256, unroll=UN)
                def _zero(r):
                    hist_v[r, :] = zero16

                _prefix, _himask = prefix, himask

                @plsc.parallel_loop(0, F, unroll=UN)
                def _count(r, p=p, shift=shift):
                    a = plsc.load_gather(amp_v, [lane_base + r])
                    u = plsc.bitcast(a, jnp.int32)
                    bk = (u >> shift) & 255
                    if p == 0:
                        plsc.addupdate_scatter(hist_v, [bk, lanes], ones)
                    else:
                        m = (u & _himask) == _prefix
                        plsc.addupdate_scatter(hist_v, [bk, lanes], ones,
                                               mask=m)

                @plsc.parallel_loop(0, 256, unroll=UN,
                                    carry=(zero16, need, prefix, zero16 > 0))
                def _scan(r, sc, shift=shift):
                    acc, need, prefix, done = sc
                    jj = 255 - r
                    h = hist_v[jj, :]
                    accn = acc + h
                    selm = jnp.logical_and(jnp.logical_not(done),
                                           accn >= need)
                    prefix = jnp.where(selm, prefix | (jj << shift), prefix)
                    need = jnp.where(selm, need - acc, need)
                    done = jnp.logical_or(done, selm)
                    return (accn, need, prefix, done)
                _, need, prefix, _ = _scan

            thr_v[...] = plsc.bitcast(prefix, jnp.float32)
            pltpu.sync_copy(thr_v, thr_hbm.at[b, pl.ds(d0, _CTILES)])
            return 0

        lax.fori_loop(0, per_w, do_block, 0)

    return select


def kernel(x):
    B, _, D = x.shape
    c = _constants()
    xr, xi, amp = _forward(x, c)
    thr = _make_select(B, D)(amp)
    rec, res = _inverse(x, xr, xi, thr, c)
    return (rec, res)


# speedup vs baseline: 6.5415x; 1.2629x over previous
import functools

import numpy as np
import jax
import jax.numpy as jnp
from jax import lax
from jax.experimental import pallas as pl
from jax.experimental.pallas import tpu as pltpu
from jax.experimental.pallas import tpu_sc as plsc

N = 8192
N1 = 64
N2 = 128
K2 = 65
F = K2 * N1
FV = N // 2 + 1
KTOP = 64
EPS = 1e-6
DBLK = 128

_HIGH = lax.Precision.HIGHEST


def _constants():
    n1 = np.arange(N1)
    k1 = np.arange(N1)
    n2 = np.arange(N2)
    k2 = np.arange(K2)
    ang1 = -2.0 * np.pi * np.outer(k1, n1) / N1
    w1r, w1i = np.cos(ang1), np.sin(ang1)
    angt = -2.0 * np.pi * np.outer(k1, n2) / N
    tr, ti = np.cos(angt), np.sin(angt)
    ange = -2.0 * np.pi * np.outer(k2, n2) / N2
    er, ei = np.cos(ange), np.sin(ange)
    b = np.arange(N2)
    a = np.arange(N1)
    ang2 = 2.0 * np.pi * np.outer(b, k2) / N2
    e2r, e2i = np.cos(ang2), np.sin(ang2)
    angw = 2.0 * np.pi * np.outer(b, k1) / N
    twr, twi = np.cos(angw), np.sin(angw)
    ang3 = 2.0 * np.pi * np.outer(a, k1) / N1
    e1r, e1i = np.cos(ang3), np.sin(ang3)
    k = (k2[:, None] * N1 + k1[None, :])
    wg = np.where(k > N // 2, 0.0, np.where((k == 0) | (k == N // 2), 1.0, 2.0)) / N
    c = lambda m: jnp.asarray(m, jnp.float32)
    return {
        "w1r": c(w1r), "w1i": c(w1i), "tr": c(tr), "ti": c(ti),
        "er": c(er), "ei": c(ei),
        "e2r": c(e2r), "e2i": c(e2i), "twr": c(twr), "twi": c(twi),
        "e1r": c(e1r), "e1i": c(e1i), "wg": c(wg),
    }


def _dot(a, b):
    return jnp.dot(a, b, preferred_element_type=jnp.float32, precision=_HIGH)


def _fwd_body(x_ref, w1r, w1i, tr, ti, er, ei, xr_out, xi_out, amp_out):
    xb = x_ref[0]
    x2 = xb.reshape(N1, N2 * DBLK)
    yr = _dot(w1r[...], x2).reshape(N1, N2, DBLK)
    yi = _dot(w1i[...], x2).reshape(N1, N2, DBLK)
    trc = tr[...][:, :, None]
    tic = ti[...][:, :, None]
    zr = yr * trc - yi * tic
    zi = yr * tic + yi * trc
    zr = jnp.swapaxes(zr, 0, 1).reshape(N2, N1 * DBLK)
    zi = jnp.swapaxes(zi, 0, 1).reshape(N2, N1 * DBLK)
    erc, eic = er[...], ei[...]
    xr = _dot(erc, zr) - _dot(eic, zi)
    xi = _dot(erc, zi) + _dot(eic, zr)
    xr = xr.reshape(F, DBLK)
    xi = xi.reshape(F, DBLK)
    amp = jnp.sqrt(jnp.square(xr + EPS) + jnp.square(xi + EPS))
    row = lax.broadcasted_iota(jnp.int32, (F, DBLK), 0)
    amp = jnp.where(row < FV, amp, 0.0)
    xr_out[0] = xr
    xi_out[0] = xi
    amp_out[0] = jnp.transpose(amp.reshape(F, DBLK // 16, 16),
                               (1, 0, 2)).reshape(DBLK // 16, F * 16)


def _forward(x, c, interpret=False):
    B, _, D = x.shape
    grid = (B, D // DBLK)
    spec_small = lambda s: pl.BlockSpec(s, lambda b, j: (0,) * len(s))
    out = pl.pallas_call(
        _fwd_body,
        grid=grid,
        in_specs=[
            pl.BlockSpec((1, N, DBLK), lambda b, j: (b, 0, j)),
            spec_small((N1, N1)), spec_small((N1, N1)),
            spec_small((N1, N2)), spec_small((N1, N2)),
            spec_small((K2, N2)), spec_small((K2, N2)),
        ],
        out_specs=[pl.BlockSpec((1, F, DBLK), lambda b, j: (b, 0, j))] * 2
        + [pl.BlockSpec((1, DBLK // 16, F * 16), lambda b, j: (b, j, 0))],
        out_shape=[jax.ShapeDtypeStruct((B, F, D), jnp.float32)] * 2
        + [jax.ShapeDtypeStruct((B, D // 16, F * 16), jnp.float32)],
        interpret=interpret,
    )(x, c["w1r"], c["w1i"], c["tr"], c["ti"], c["er"], c["ei"])
    return out


def _inv_body(x_ref, xr_ref, xi_ref, thr_ref,
              e2r, e2i, twr, twi, e1r, e1i, wg,
              rec_out, res_out):
    xr = xr_ref[0]
    xi = xi_ref[0]
    amp = jnp.sqrt(jnp.square(xr + EPS) + jnp.square(xi + EPS))
    thr = thr_ref[0]
    keep = (amp >= thr).astype(jnp.float32)
    wgc = wg[...].reshape(F, 1)
    yhr = (xr * keep * wgc).reshape(K2, N1 * DBLK)
    yhi = (xi * keep * wgc).reshape(K2, N1 * DBLK)
    e2rc, e2ic = e2r[...], e2i[...]
    ur = (_dot(e2rc, yhr) - _dot(e2ic, yhi)).reshape(N2, N1, DBLK)
    ui = (_dot(e2rc, yhi) + _dot(e2ic, yhr)).reshape(N2, N1, DBLK)
    twrc = twr[...][:, :, None]
    twic = twi[...][:, :, None]
    vr = ur * twrc - ui * twic
    vi = ur * twic + ui * twrc
    vr = jnp.swapaxes(vr, 0, 1).reshape(N1, N2 * DBLK)
    vi = jnp.swapaxes(vi, 0, 1).reshape(N1, N2 * DBLK)
    rec = (_dot(e1r[...], vr) - _dot(e1i[...], vi)).reshape(N, DBLK)
    rec_out[0] = rec
    res_out[0] = x_ref[0] - rec


def _inverse(x, xr, xi, thr, c, interpret=False):
    B, _, D = x.shape
    grid = (B, D // DBLK)
    spec_small = lambda s: pl.BlockSpec(s, lambda b, j: (0,) * len(s))
    rec, res = pl.pallas_call(
        _inv_body,
        grid=grid,
        in_specs=[
            pl.BlockSpec((1, N, DBLK), lambda b, j: (b, 0, j)),
            pl.BlockSpec((1, F, DBLK), lambda b, j: (b, 0, j)),
            pl.BlockSpec((1, F, DBLK), lambda b, j: (b, 0, j)),
            pl.BlockSpec((1, 1, DBLK), lambda b, j: (b, 0, j)),
            spec_small((N2, K2)), spec_small((N2, K2)),
            spec_small((N2, N1)), spec_small((N2, N1)),
            spec_small((N1, N1)), spec_small((N1, N1)),
            spec_small((F,)),
        ],
        out_specs=[pl.BlockSpec((1, N, DBLK), lambda b, j: (b, 0, j))] * 2,
        out_shape=[jax.ShapeDtypeStruct((B, N, D), jnp.float32)] * 2,
        interpret=interpret,
    )(x, xr, xi, thr.reshape(B, 1, D), c["e2r"], c["e2i"], c["twr"], c["twi"],
      c["e1r"], c["e1i"], c["wg"].reshape(F))
    return rec, res


_CTILES = 16


def _make_select(B, D):
    nblocks = (B * D) // _CTILES
    nworkers = 32
    per_w = nblocks // nworkers
    dtiles = D // _CTILES

    mesh = plsc.VectorSubcoreMesh(core_axis_name="c", subcore_axis_name="s")

    @functools.partial(
        pl.kernel,
        out_type=jax.ShapeDtypeStruct((B, D), jnp.float32),
        mesh=mesh,
        compiler_params=pltpu.CompilerParams(
            use_tc_tiling_on_sc=False, needs_layout_passes=False),
        scratch_types=[
            pltpu.VMEM((F * _CTILES,), jnp.float32),
            pltpu.VMEM((256, _CTILES), jnp.int32),
            pltpu.VMEM((_CTILES,), jnp.float32),
            pltpu.SemaphoreType.DMA,
        ],
    )
    def select(amp_hbm, thr_hbm, amp_v, hist_v, thr_v, dsem):
        wid = lax.axis_index("s") * 2 + lax.axis_index("c")
        lanes = lax.iota(jnp.int32, 16)
        ones = jnp.full((16,), 1, jnp.int32)
        zero16 = jnp.zeros((16,), jnp.int32)
        UN = 8

        def do_block(i, _):
            g = wid * per_w + i
            b = g // dtiles
            d0 = (g % dtiles) * _CTILES
            pltpu.sync_copy(amp_hbm.at[b, g % dtiles, :], amp_v)

            prefix = zero16
            need = jnp.full((16,), KTOP, jnp.int32)
            for p in range(4):
                shift = 24 - 8 * p
                hm_py = (0xFFFFFFFF << (32 - 8 * p)) & 0xFFFFFFFF if p else 0
                himask = jnp.int32(hm_py - (1 << 32) if hm_py >= (1 << 31) else hm_py)

                @plsc.parallel_loop(0, 256, unroll=UN)
                def _zero(r):
                    hist_v[r, :] = zero16

                _prefix, _himask = prefix, himask

                @plsc.parallel_loop(0, F, unroll=UN)
                def _count(r, p=p, shift=shift):
                    u = plsc.bitcast(amp_v[pl.ds(r * 16, 16)], jnp.int32)
                    bk = (u >> shift) & 255
                    if p == 0:
                        plsc.addupdate_scatter(hist_v, [bk, lanes], ones)
                    else:
                        m = (u & _himask) == _prefix
                        plsc.addupdate_scatter(hist_v, [bk, lanes], ones,
                                               mask=m)

                @plsc.parallel_loop(0, 256, unroll=UN,
                                    carry=(zero16, need, prefix, zero16 > 0))
                def _scan(r, sc, shift=shift):
                    acc, need, prefix, done = sc
                    jj = 255 - r
                    h = hist_v[jj, :]
                    accn = acc + h
                    selm = jnp.logical_and(jnp.logical_not(done),
                                           accn >= need)
                    prefix = jnp.where(selm, prefix | (jj << shift), prefix)
                    need = jnp.where(selm, need - acc, need)
                    done = jnp.logical_or(done, selm)
                    return (accn, need, prefix, done)
                _, need, prefix, _ = _scan

            thr_v[...] = plsc.bitcast(prefix, jnp.float32)
            pltpu.sync_copy(thr_v, thr_hbm.at[b, pl.ds(d0, _CTILES)])
            return 0

        lax.fori_loop(0, per_w, do_block, 0)

    return select


def kernel(x):
    B, _, D = x.shape
    c = _constants()
    xr, xi, amp = _forward(x, c)
    thr = _make_select(B, D)(amp)
    rec, res = _inverse(x, xr, xi, thr, c)
    return (rec, res)


# speedup vs baseline: 7.5617x; 1.1560x over previous
import functools

import numpy as np
import jax
import jax.numpy as jnp
from jax import lax
from jax.experimental import pallas as pl
from jax.experimental.pallas import tpu as pltpu
from jax.experimental.pallas import tpu_sc as plsc

N = 8192
N1 = 64
N2 = 128
K2 = 65
F = K2 * N1
FV = N // 2 + 1
KTOP = 64
EPS = 1e-6
DBLK = 128

_HIGH = lax.Precision.HIGHEST


def _constants():
    n1 = np.arange(N1)
    k1 = np.arange(N1)
    n2 = np.arange(N2)
    k2 = np.arange(K2)
    ang1 = -2.0 * np.pi * np.outer(k1, n1) / N1
    w1r, w1i = np.cos(ang1), np.sin(ang1)
    angt = -2.0 * np.pi * np.outer(k1, n2) / N
    tr, ti = np.cos(angt), np.sin(angt)
    ange = -2.0 * np.pi * np.outer(k2, n2) / N2
    er, ei = np.cos(ange), np.sin(ange)
    b = np.arange(N2)
    a = np.arange(N1)
    ang2 = 2.0 * np.pi * np.outer(b, k2) / N2
    e2r, e2i = np.cos(ang2), np.sin(ang2)
    angw = 2.0 * np.pi * np.outer(b, k1) / N
    twr, twi = np.cos(angw), np.sin(angw)
    ang3 = 2.0 * np.pi * np.outer(a, k1) / N1
    e1r, e1i = np.cos(ang3), np.sin(ang3)
    k = (k2[:, None] * N1 + k1[None, :])
    wg = np.where(k > N // 2, 0.0, np.where((k == 0) | (k == N // 2), 1.0, 2.0)) / N
    c = lambda m: jnp.asarray(m, jnp.float32)
    return {
        "w1r": c(w1r), "w1i": c(w1i), "tr": c(tr), "ti": c(ti),
        "er": c(er), "ei": c(ei),
        "e2r": c(e2r), "e2i": c(e2i), "twr": c(twr), "twi": c(twi),
        "e1r": c(e1r), "e1i": c(e1i), "wg": c(wg),
    }


def _dot(a, b):
    return jnp.dot(a, b, preferred_element_type=jnp.float32, precision=_HIGH)


def _fwd_body(x_ref, w1r, w1i, tr, ti, er, ei, xr_out, xi_out, amp_out):
    xb = x_ref[0]
    x2 = xb.reshape(N1, N2 * DBLK)
    yr = _dot(w1r[...], x2).reshape(N1, N2, DBLK)
    yi = _dot(w1i[...], x2).reshape(N1, N2, DBLK)
    trc = tr[...][:, :, None]
    tic = ti[...][:, :, None]
    zr = yr * trc - yi * tic
    zi = yr * tic + yi * trc
    zr = jnp.swapaxes(zr, 0, 1).reshape(N2, N1 * DBLK)
    zi = jnp.swapaxes(zi, 0, 1).reshape(N2, N1 * DBLK)
    erc, eic = er[...], ei[...]
    xr = _dot(erc, zr) - _dot(eic, zi)
    xi = _dot(erc, zi) + _dot(eic, zr)
    xr = xr.reshape(F, DBLK)
    xi = xi.reshape(F, DBLK)
    amp = jnp.sqrt(jnp.square(xr + EPS) + jnp.square(xi + EPS))
    row = lax.broadcasted_iota(jnp.int32, (F, DBLK), 0)
    amp = jnp.where(row < FV, amp, 0.0)
    xr_out[0] = xr
    xi_out[0] = xi
    amp_out[0] = amp


def _forward(x, c, interpret=False):
    B, _, D = x.shape
    grid = (B, D // DBLK)
    spec_small = lambda s: pl.BlockSpec(s, lambda b, j: (0,) * len(s))
    out = pl.pallas_call(
        _fwd_body,
        grid=grid,
        in_specs=[
            pl.BlockSpec((1, N, DBLK), lambda b, j: (b, 0, j)),
            spec_small((N1, N1)), spec_small((N1, N1)),
            spec_small((N1, N2)), spec_small((N1, N2)),
            spec_small((K2, N2)), spec_small((K2, N2)),
        ],
        out_specs=[pl.BlockSpec((1, F, DBLK), lambda b, j: (b, 0, j))] * 3,
        out_shape=[jax.ShapeDtypeStruct((B, F, D), jnp.float32)] * 3,
        interpret=interpret,
    )(x, c["w1r"], c["w1i"], c["tr"], c["ti"], c["er"], c["ei"])
    return out


def _inv_body(x_ref, xr_ref, xi_ref, thr_ref,
              e2r, e2i, twr, twi, e1r, e1i, wg,
              rec_out, res_out):
    xr = xr_ref[0]
    xi = xi_ref[0]
    amp = jnp.sqrt(jnp.square(xr + EPS) + jnp.square(xi + EPS))
    thr = thr_ref[0]
    keep = (amp >= thr).astype(jnp.float32)
    wgc = wg[...].reshape(F, 1)
    yhr = (xr * keep * wgc).reshape(K2, N1 * DBLK)
    yhi = (xi * keep * wgc).reshape(K2, N1 * DBLK)
    e2rc, e2ic = e2r[...], e2i[...]
    ur = (_dot(e2rc, yhr) - _dot(e2ic, yhi)).reshape(N2, N1, DBLK)
    ui = (_dot(e2rc, yhi) + _dot(e2ic, yhr)).reshape(N2, N1, DBLK)
    twrc = twr[...][:, :, None]
    twic = twi[...][:, :, None]
    vr = ur * twrc - ui * twic
    vi = ur * twic + ui * twrc
    vr = jnp.swapaxes(vr, 0, 1).reshape(N1, N2 * DBLK)
    vi = jnp.swapaxes(vi, 0, 1).reshape(N1, N2 * DBLK)
    rec = (_dot(e1r[...], vr) - _dot(e1i[...], vi)).reshape(N, DBLK)
    rec_out[0] = rec
    res_out[0] = x_ref[0] - rec


def _inverse(x, xr, xi, thr, c, interpret=False):
    B, _, D = x.shape
    grid = (B, D // DBLK)
    spec_small = lambda s: pl.BlockSpec(s, lambda b, j: (0,) * len(s))
    rec, res = pl.pallas_call(
        _inv_body,
        grid=grid,
        in_specs=[
            pl.BlockSpec((1, N, DBLK), lambda b, j: (b, 0, j)),
            pl.BlockSpec((1, F, DBLK), lambda b, j: (b, 0, j)),
            pl.BlockSpec((1, F, DBLK), lambda b, j: (b, 0, j)),
            pl.BlockSpec((1, 1, DBLK), lambda b, j: (b, 0, j)),
            spec_small((N2, K2)), spec_small((N2, K2)),
            spec_small((N2, N1)), spec_small((N2, N1)),
            spec_small((N1, N1)), spec_small((N1, N1)),
            spec_small((F,)),
        ],
        out_specs=[pl.BlockSpec((1, N, DBLK), lambda b, j: (b, 0, j))] * 2,
        out_shape=[jax.ShapeDtypeStruct((B, N, D), jnp.float32)] * 2,
        interpret=interpret,
    )(x, xr, xi, thr.reshape(B, 1, D), c["e2r"], c["e2i"], c["twr"], c["twi"],
      c["e1r"], c["e1i"], c["wg"].reshape(F))
    return rec, res


_CTILES = 16


def _make_select(B, D):
    nblocks = (B * D) // _CTILES
    nworkers = 32
    per_w = nblocks // nworkers
    dtiles = D // _CTILES

    mesh = plsc.VectorSubcoreMesh(core_axis_name="c", subcore_axis_name="s")

    @functools.partial(
        pl.kernel,
        out_type=jax.ShapeDtypeStruct((B, D), jnp.float32),
        mesh=mesh,
        compiler_params=pltpu.CompilerParams(
            use_tc_tiling_on_sc=False, needs_layout_passes=False),
        scratch_types=[
            pltpu.VMEM((F, _CTILES), jnp.float32),
            pltpu.VMEM((256, _CTILES), jnp.int32),
            pltpu.VMEM((_CTILES,), jnp.float32),
            pltpu.SemaphoreType.DMA,
        ],
    )
    def select(amp_hbm, thr_hbm, amp_v, hist_v, thr_v, dsem):
        wid = lax.axis_index("s") * 2 + lax.axis_index("c")
        lanes = lax.iota(jnp.int32, 16)
        ones = jnp.full((16,), 1, jnp.int32)
        zero16 = jnp.zeros((16,), jnp.int32)
        UN = 8

        def do_block(i, _):
            g = wid * per_w + i
            b = g // dtiles
            d0 = (g % dtiles) * _CTILES
            pltpu.sync_copy(amp_hbm.at[b, :, pl.ds(d0, _CTILES)], amp_v)

            prefix = zero16
            need = jnp.full((16,), KTOP, jnp.int32)
            for p in range(4):
                shift = 24 - 8 * p
                hm_py = (0xFFFFFFFF << (32 - 8 * p)) & 0xFFFFFFFF if p else 0
                himask = jnp.int32(hm_py - (1 << 32) if hm_py >= (1 << 31) else hm_py)

                @plsc.parallel_loop(0, 256, unroll=UN)
                def _zero(r):
                    hist_v[r, :] = zero16

                _prefix, _himask = prefix, himask

                @plsc.parallel_loop(0, F, unroll=UN)
                def _count(r, p=p, shift=shift):
                    u = plsc.bitcast(amp_v[r, :], jnp.int32)
                    bk = (u >> shift) & 255
                    if p == 0:
                        plsc.addupdate_scatter(hist_v, [bk, lanes], ones)
                    else:
                        m = (u & _himask) == _prefix
                        plsc.addupdate_scatter(hist_v, [bk, lanes], ones,
                                               mask=m)

                @plsc.parallel_loop(0, 256, unroll=UN,
                                    carry=(zero16, need, prefix, zero16 > 0))
                def _scan(r, sc, shift=shift):
                    acc, need, prefix, done = sc
                    jj = 255 - r
                    h = hist_v[jj, :]
                    accn = acc + h
                    selm = jnp.logical_and(jnp.logical_not(done),
                                           accn >= need)
                    prefix = jnp.where(selm, prefix | (jj << shift), prefix)
                    need = jnp.where(selm, need - acc, need)
                    done = jnp.logical_or(done, selm)
                    return (accn, need, prefix, done)
                _, need, prefix, _ = _scan

            thr_v[...] = plsc.bitcast(prefix, jnp.float32)
            pltpu.sync_copy(thr_v, thr_hbm.at[b, pl.ds(d0, _CTILES)])
            return 0

        lax.fori_loop(0, per_w, do_block, 0)

    return select


def kernel(x):
    B, _, D = x.shape
    c = _constants()
    xr, xi, amp = _forward(x, c)
    thr = _make_select(B, D)(amp)
    rec, res = _inverse(x, xr, xi, thr, c)
    return (rec, res)


# speedup vs baseline: 8.1601x; 1.0791x over previous
import functools

import numpy as np
import jax
import jax.numpy as jnp
from jax import lax
from jax.experimental import pallas as pl
from jax.experimental.pallas import tpu as pltpu
from jax.experimental.pallas import tpu_sc as plsc

N = 8192
N1 = 64
N2 = 128
K2 = 65
F = K2 * N1
FV = N // 2 + 1
KTOP = 64
EPS = 1e-6
DBLK = 128

_HIGH = lax.Precision.HIGHEST


def _constants():
    n1 = np.arange(N1)
    k1 = np.arange(N1)
    n2 = np.arange(N2)
    k2 = np.arange(K2)
    ang1 = -2.0 * np.pi * np.outer(k1, n1) / N1
    w1r, w1i = np.cos(ang1), np.sin(ang1)
    angt = -2.0 * np.pi * np.outer(k1, n2) / N
    tr, ti = np.cos(angt), np.sin(angt)
    ange = -2.0 * np.pi * np.outer(k2, n2) / N2
    er, ei = np.cos(ange), np.sin(ange)
    b = np.arange(N2)
    a = np.arange(N1)
    ang2 = 2.0 * np.pi * np.outer(b, k2) / N2
    e2r, e2i = np.cos(ang2), np.sin(ang2)
    angw = 2.0 * np.pi * np.outer(b, k1) / N
    twr, twi = np.cos(angw), np.sin(angw)
    ang3 = 2.0 * np.pi * np.outer(a, k1) / N1
    e1r, e1i = np.cos(ang3), np.sin(ang3)
    k = (k2[:, None] * N1 + k1[None, :])
    wg = np.where(k > N // 2, 0.0, np.where((k == 0) | (k == N // 2), 1.0, 2.0)) / N
    c = lambda m: jnp.asarray(m, jnp.float32)
    return {
        "w1r": c(w1r), "w1i": c(w1i), "tr": c(tr), "ti": c(ti),
        "er": c(er), "erpi": c(er + ei), "ermi": c(er - ei),
        "e2r": c(e2r), "e2pi": c(e2r + e2i), "e2mi": c(e2r - e2i),
        "twr": c(twr), "twi": c(twi),
        "e1r": c(e1r), "e1i": c(e1i), "wg": c(wg),
    }


def _dot(a, b):
    return jnp.dot(a, b, preferred_element_type=jnp.float32, precision=_HIGH)


def _fwd_body(x_ref, w1r, w1i, tr, ti, er, erpi, ermi, xr_out, xi_out, amp_out):
    xb = x_ref[0]
    x2 = xb.reshape(N1, N2 * DBLK)
    yr = _dot(w1r[...], x2).reshape(N1, N2, DBLK)
    yi = _dot(w1i[...], x2).reshape(N1, N2, DBLK)
    trc = tr[...][:, :, None]
    tic = ti[...][:, :, None]
    zr = yr * trc - yi * tic
    zi = yr * tic + yi * trc
    zr = jnp.swapaxes(zr, 0, 1).reshape(N2, N1 * DBLK)
    zi = jnp.swapaxes(zi, 0, 1).reshape(N2, N1 * DBLK)
    pa = _dot(er[...], zr + zi)
    xr = pa - _dot(erpi[...], zi)
    xi = pa - _dot(ermi[...], zr)
    xr = xr.reshape(F, DBLK)
    xi = xi.reshape(F, DBLK)
    amp = jnp.sqrt(jnp.square(xr + EPS) + jnp.square(xi + EPS))
    row = lax.broadcasted_iota(jnp.int32, (F, DBLK), 0)
    amp = jnp.where(row < FV, amp, 0.0)
    xr_out[0] = xr
    xi_out[0] = xi
    amp_out[0] = amp


def _forward(x, c, interpret=False):
    B, _, D = x.shape
    grid = (B, D // DBLK)
    spec_small = lambda s: pl.BlockSpec(s, lambda b, j: (0,) * len(s))
    out = pl.pallas_call(
        _fwd_body,
        grid=grid,
        in_specs=[
            pl.BlockSpec((1, N, DBLK), lambda b, j: (b, 0, j)),
            spec_small((N1, N1)), spec_small((N1, N1)),
            spec_small((N1, N2)), spec_small((N1, N2)),
            spec_small((K2, N2)), spec_small((K2, N2)), spec_small((K2, N2)),
        ],
        out_specs=[pl.BlockSpec((1, F, DBLK), lambda b, j: (b, 0, j))] * 3,
        out_shape=[jax.ShapeDtypeStruct((B, F, D), jnp.float32)] * 3,
        interpret=interpret,
    )(x, c["w1r"], c["w1i"], c["tr"], c["ti"], c["er"], c["erpi"], c["ermi"])
    return out


def _inv_body(x_ref, xr_ref, xi_ref, thr_ref,
              e2r, e2pi, e2mi, twr, twi, e1r, e1i, wg,
              rec_out, res_out):
    xr = xr_ref[0]
    xi = xi_ref[0]
    amp = jnp.sqrt(jnp.square(xr + EPS) + jnp.square(xi + EPS))
    thr = thr_ref[0]
    keep = (amp >= thr).astype(jnp.float32)
    wgc = wg[...].reshape(F, 1)
    yhr = (xr * keep * wgc).reshape(K2, N1 * DBLK)
    yhi = (xi * keep * wgc).reshape(K2, N1 * DBLK)
    pa = _dot(e2r[...], yhr + yhi)
    ur = (pa - _dot(e2pi[...], yhi)).reshape(N2, N1, DBLK)
    ui = (pa - _dot(e2mi[...], yhr)).reshape(N2, N1, DBLK)
    twrc = twr[...][:, :, None]
    twic = twi[...][:, :, None]
    vr = ur * twrc - ui * twic
    vi = ur * twic + ui * twrc
    vr = jnp.swapaxes(vr, 0, 1).reshape(N1, N2 * DBLK)
    vi = jnp.swapaxes(vi, 0, 1).reshape(N1, N2 * DBLK)
    rec = (_dot(e1r[...], vr) - _dot(e1i[...], vi)).reshape(N, DBLK)
    rec_out[0] = rec
    res_out[0] = x_ref[0] - rec


def _inverse(x, xr, xi, thr, c, interpret=False):
    B, _, D = x.shape
    grid = (B, D // DBLK)
    spec_small = lambda s: pl.BlockSpec(s, lambda b, j: (0,) * len(s))
    rec, res = pl.pallas_call(
        _inv_body,
        grid=grid,
        in_specs=[
            pl.BlockSpec((1, N, DBLK), lambda b, j: (b, 0, j)),
            pl.BlockSpec((1, F, DBLK), lambda b, j: (b, 0, j)),
            pl.BlockSpec((1, F, DBLK), lambda b, j: (b, 0, j)),
            pl.BlockSpec((1, 1, DBLK), lambda b, j: (b, 0, j)),
            spec_small((N2, K2)), spec_small((N2, K2)), spec_small((N2, K2)),
            spec_small((N2, N1)), spec_small((N2, N1)),
            spec_small((N1, N1)), spec_small((N1, N1)),
            spec_small((F,)),
        ],
        out_specs=[pl.BlockSpec((1, N, DBLK), lambda b, j: (b, 0, j))] * 2,
        out_shape=[jax.ShapeDtypeStruct((B, N, D), jnp.float32)] * 2,
        interpret=interpret,
    )(x, xr, xi, thr.reshape(B, 1, D), c["e2r"], c["e2pi"], c["e2mi"],
      c["twr"], c["twi"],
      c["e1r"], c["e1i"], c["wg"].reshape(F))
    return rec, res


_CTILES = 16


def _make_select(B, D):
    nblocks = (B * D) // _CTILES
    nworkers = 32
    per_w = nblocks // nworkers
    dtiles = D // _CTILES

    mesh = plsc.VectorSubcoreMesh(core_axis_name="c", subcore_axis_name="s")

    @functools.partial(
        pl.kernel,
        out_type=jax.ShapeDtypeStruct((B, D), jnp.float32),
        mesh=mesh,
        compiler_params=pltpu.CompilerParams(
            use_tc_tiling_on_sc=False, needs_layout_passes=False),
        scratch_types=[
            pltpu.VMEM((F, _CTILES), jnp.float32),
            pltpu.VMEM((256, _CTILES), jnp.int32),
            pltpu.VMEM((_CTILES,), jnp.float32),
            pltpu.SemaphoreType.DMA,
        ],
    )
    def select(amp_hbm, thr_hbm, amp_v, hist_v, thr_v, dsem):
        wid = lax.axis_index("s") * 2 + lax.axis_index("c")
        lanes = lax.iota(jnp.int32, 16)
        ones = jnp.full((16,), 1, jnp.int32)
        zero16 = jnp.zeros((16,), jnp.int32)
        UN = 8

        def do_block(i, _):
            g = wid * per_w + i
            b = g // dtiles
            d0 = (g % dtiles) * _CTILES
            pltpu.sync_copy(amp_hbm.at[b, :, pl.ds(d0, _CTILES)], amp_v)

            prefix = zero16
            need = jnp.full((16,), KTOP, jnp.int32)
            for p in range(4):
                shift = 24 - 8 * p
                hm_py = (0xFFFFFFFF << (32 - 8 * p)) & 0xFFFFFFFF if p else 0
                himask = jnp.int32(hm_py - (1 << 32) if hm_py >= (1 << 31) else hm_py)

                @plsc.parallel_loop(0, 256, unroll=UN)
                def _zero(r):
                    hist_v[r, :] = zero16

                _prefix, _himask = prefix, himask

                @plsc.parallel_loop(0, F, unroll=UN)
                def _count(r, p=p, shift=shift):
                    u = plsc.bitcast(amp_v[r, :], jnp.int32)
                    bk = (u >> shift) & 255
                    if p == 0:
                        plsc.addupdate_scatter(hist_v, [bk, lanes], ones)
                    else:
                        m = (u & _himask) == _prefix
                        plsc.addupdate_scatter(hist_v, [bk, lanes], ones,
                                               mask=m)

                @plsc.parallel_loop(0, 256, unroll=UN,
                                    carry=(zero16, need, prefix, zero16 > 0))
                def _scan(r, sc, shift=shift):
                    acc, need, prefix, done = sc
                    jj = 255 - r
                    h = hist_v[jj, :]
                    accn = acc + h
                    selm = jnp.logical_and(jnp.logical_not(done),
                                           accn >= need)
                    prefix = jnp.where(selm, prefix | (jj << shift), prefix)
                    need = jnp.where(selm, need - acc, need)
                    done = jnp.logical_or(done, selm)
                    return (accn, need, prefix, done)
                _, need, prefix, _ = _scan

            thr_v[...] = plsc.bitcast(prefix, jnp.float32)
            pltpu.sync_copy(thr_v, thr_hbm.at[b, pl.ds(d0, _CTILES)])
            return 0

        lax.fori_loop(0, per_w, do_block, 0)

    return select


def kernel(x):
    B, _, D = x.shape
    c = _constants()
    xr, xi, amp = _forward(x, c)
    thr = _make_select(B, D)(amp)
    rec, res = _inverse(x, xr, xi, thr, c)
    return (rec, res)


# speedup vs baseline: 8.4814x; 1.0394x over previous
import functools

import numpy as np
import jax
import jax.numpy as jnp
from jax import lax
from jax.experimental import pallas as pl
from jax.experimental.pallas import tpu as pltpu
from jax.experimental.pallas import tpu_sc as plsc

N = 8192
N1 = 64
N2 = 128
K2 = 65
F = K2 * N1
FV = N // 2 + 1
KTOP = 64
EPS = 1e-6
DBLK = 128

_HIGH = lax.Precision.HIGHEST


def _constants():
    n1 = np.arange(N1)
    k1 = np.arange(N1)
    n2 = np.arange(N2)
    k2 = np.arange(K2)
    ang1 = -2.0 * np.pi * np.outer(k1, n1) / N1
    w1r, w1i = np.cos(ang1), np.sin(ang1)
    angt = -2.0 * np.pi * np.outer(k1, n2) / N
    tr, ti = np.cos(angt), np.sin(angt)
    ange = -2.0 * np.pi * np.outer(k2, n2) / N2
    er, ei = np.cos(ange), np.sin(ange)
    b = np.arange(N2)
    a = np.arange(N1)
    ang2 = 2.0 * np.pi * np.outer(b, k2) / N2
    e2r, e2i = np.cos(ang2), np.sin(ang2)
    angw = 2.0 * np.pi * np.outer(b, k1) / N
    twr, twi = np.cos(angw), np.sin(angw)
    ang3 = 2.0 * np.pi * np.outer(a, k1) / N1
    e1r, e1i = np.cos(ang3), np.sin(ang3)
    k = (k2[:, None] * N1 + k1[None, :])
    wg = np.where(k > N // 2, 0.0, np.where((k == 0) | (k == N // 2), 1.0, 2.0)) / N
    c = lambda m: jnp.asarray(m, jnp.float32)
    return {
        "w1r": c(w1r), "w1i": c(w1i), "tr": c(tr), "ti": c(ti),
        "er": c(er), "erpi": c(er + ei), "ermi": c(er - ei),
        "e2r": c(e2r), "e2pi": c(e2r + e2i), "e2mi": c(e2r - e2i),
        "twr": c(twr), "twi": c(twi),
        "e1r": c(e1r), "e1i": c(e1i), "wg": c(wg),
    }


def _dot(a, b):
    return jnp.dot(a, b, preferred_element_type=jnp.float32, precision=_HIGH)


def _fwd_body(x_ref, w1r, w1i, tr, ti, er, erpi, ermi, xr_out, xi_out, amp_out):
    xb = x_ref[0]
    x2 = xb.reshape(N1, N2 * DBLK)
    yr = _dot(w1r[...], x2).reshape(N1, N2, DBLK)
    yi = _dot(w1i[...], x2).reshape(N1, N2, DBLK)
    trc = tr[...][:, :, None]
    tic = ti[...][:, :, None]
    zr = yr * trc - yi * tic
    zi = yr * tic + yi * trc
    zr = jnp.swapaxes(zr, 0, 1).reshape(N2, N1 * DBLK)
    zi = jnp.swapaxes(zi, 0, 1).reshape(N2, N1 * DBLK)
    pa = _dot(er[...], zr + zi)
    xr = pa - _dot(erpi[...], zi)
    xi = pa - _dot(ermi[...], zr)
    xr = xr.reshape(F, DBLK)
    xi = xi.reshape(F, DBLK)
    amp = jnp.square(xr + EPS) + jnp.square(xi + EPS)
    row = lax.broadcasted_iota(jnp.int32, (F, DBLK), 0)
    amp = jnp.where(row < FV, amp, 0.0)
    xr_out[0] = xr
    xi_out[0] = xi
    amp_out[0] = amp


def _forward(x, c, interpret=False):
    B, _, D = x.shape
    grid = (B, D // DBLK)
    spec_small = lambda s: pl.BlockSpec(s, lambda b, j: (0,) * len(s))
    out = pl.pallas_call(
        _fwd_body,
        grid=grid,
        in_specs=[
            pl.BlockSpec((1, N, DBLK), lambda b, j: (b, 0, j)),
            spec_small((N1, N1)), spec_small((N1, N1)),
            spec_small((N1, N2)), spec_small((N1, N2)),
            spec_small((K2, N2)), spec_small((K2, N2)), spec_small((K2, N2)),
        ],
        out_specs=[pl.BlockSpec((1, F, DBLK), lambda b, j: (b, 0, j))] * 3,
        out_shape=[jax.ShapeDtypeStruct((B, F, D), jnp.float32)] * 3,
        interpret=interpret,
    )(x, c["w1r"], c["w1i"], c["tr"], c["ti"], c["er"], c["erpi"], c["ermi"])
    return out


def _inv_body(x_ref, xr_ref, xi_ref, thr_ref,
              e2r, e2pi, e2mi, twr, twi, e1r, e1i, wg,
              rec_out, res_out):
    xr = xr_ref[0]
    xi = xi_ref[0]
    amp = jnp.square(xr + EPS) + jnp.square(xi + EPS)
    thr = thr_ref[0]
    keep = (amp >= thr).astype(jnp.float32)
    wgc = wg[...].reshape(F, 1)
    yhr = (xr * keep * wgc).reshape(K2, N1 * DBLK)
    yhi = (xi * keep * wgc).reshape(K2, N1 * DBLK)
    pa = _dot(e2r[...], yhr + yhi)
    ur = (pa - _dot(e2pi[...], yhi)).reshape(N2, N1, DBLK)
    ui = (pa - _dot(e2mi[...], yhr)).reshape(N2, N1, DBLK)
    twrc = twr[...][:, :, None]
    twic = twi[...][:, :, None]
    vr = ur * twrc - ui * twic
    vi = ur * twic + ui * twrc
    vr = jnp.swapaxes(vr, 0, 1).reshape(N1, N2 * DBLK)
    vi = jnp.swapaxes(vi, 0, 1).reshape(N1, N2 * DBLK)
    rec = (_dot(e1r[...], vr) - _dot(e1i[...], vi)).reshape(N, DBLK)
    rec_out[0] = rec
    res_out[0] = x_ref[0] - rec


def _inverse(x, xr, xi, thr, c, interpret=False):
    B, _, D = x.shape
    grid = (B, D // DBLK)
    spec_small = lambda s: pl.BlockSpec(s, lambda b, j: (0,) * len(s))
    rec, res = pl.pallas_call(
        _inv_body,
        grid=grid,
        in_specs=[
            pl.BlockSpec((1, N, DBLK), lambda b, j: (b, 0, j)),
            pl.BlockSpec((1, F, DBLK), lambda b, j: (b, 0, j)),
            pl.BlockSpec((1, F, DBLK), lambda b, j: (b, 0, j)),
            pl.BlockSpec((1, 1, DBLK), lambda b, j: (b, 0, j)),
            spec_small((N2, K2)), spec_small((N2, K2)), spec_small((N2, K2)),
            spec_small((N2, N1)), spec_small((N2, N1)),
            spec_small((N1, N1)), spec_small((N1, N1)),
            spec_small((F,)),
        ],
        out_specs=[pl.BlockSpec((1, N, DBLK), lambda b, j: (b, 0, j))] * 2,
        out_shape=[jax.ShapeDtypeStruct((B, N, D), jnp.float32)] * 2,
        interpret=interpret,
    )(x, xr, xi, thr.reshape(B, 1, D), c["e2r"], c["e2pi"], c["e2mi"],
      c["twr"], c["twi"],
      c["e1r"], c["e1i"], c["wg"].reshape(F))
    return rec, res


_CTILES = 16


def _make_select(B, D):
    nblocks = (B * D) // _CTILES
    nworkers = 32
    per_w = nblocks // nworkers
    dtiles = D // _CTILES

    mesh = plsc.VectorSubcoreMesh(core_axis_name="c", subcore_axis_name="s")

    @functools.partial(
        pl.kernel,
        out_type=jax.ShapeDtypeStruct((B, D), jnp.float32),
        mesh=mesh,
        compiler_params=pltpu.CompilerParams(
            use_tc_tiling_on_sc=False, needs_layout_passes=False),
        scratch_types=[
            pltpu.VMEM((F, _CTILES), jnp.float32),
            pltpu.VMEM((256, _CTILES), jnp.int32),
            pltpu.VMEM((_CTILES,), jnp.float32),
            pltpu.SemaphoreType.DMA,
        ],
    )
    def select(amp_hbm, thr_hbm, amp_v, hist_v, thr_v, dsem):
        wid = lax.axis_index("s") * 2 + lax.axis_index("c")
        lanes = lax.iota(jnp.int32, 16)
        ones = jnp.full((16,), 1, jnp.int32)
        zero16 = jnp.zeros((16,), jnp.int32)
        UN = 8

        def do_block(i, _):
            g = wid * per_w + i
            b = g // dtiles
            d0 = (g % dtiles) * _CTILES
            pltpu.sync_copy(amp_hbm.at[b, :, pl.ds(d0, _CTILES)], amp_v)

            prefix = zero16
            need = jnp.full((16,), KTOP, jnp.int32)
            for p in range(4):
                shift = 24 - 8 * p
                hm_py = (0xFFFFFFFF << (32 - 8 * p)) & 0xFFFFFFFF if p else 0
                himask = jnp.int32(hm_py - (1 << 32) if hm_py >= (1 << 31) else hm_py)

                @plsc.parallel_loop(0, 256, unroll=UN)
                def _zero(r):
                    hist_v[r, :] = zero16

                _prefix, _himask = prefix, himask

                @plsc.parallel_loop(0, F, unroll=UN)
                def _count(r, p=p, shift=shift):
                    u = plsc.bitcast(amp_v[r, :], jnp.int32)
                    bk = (u >> shift) & 255
                    if p == 0:
                        plsc.addupdate_scatter(hist_v, [bk, lanes], ones)
                    else:
                        m = (u & _himask) == _prefix
                        plsc.addupdate_scatter(hist_v, [bk, lanes], ones,
                                               mask=m)

                @plsc.parallel_loop(0, 256, unroll=UN,
                                    carry=(zero16, need, prefix, zero16 > 0))
                def _scan(r, sc, shift=shift):
                    acc, need, prefix, done = sc
                    jj = 255 - r
                    h = hist_v[jj, :]
                    accn = acc + h
                    selm = jnp.logical_and(jnp.logical_not(done),
                                           accn >= need)
                    prefix = jnp.where(selm, prefix | (jj << shift), prefix)
                    need = jnp.where(selm, need - acc, need)
                    done = jnp.logical_or(done, selm)
                    return (accn, need, prefix, done)
                _, need, prefix, _ = _scan

            thr_v[...] = plsc.bitcast(prefix, jnp.float32)
            pltpu.sync_copy(thr_v, thr_hbm.at[b, pl.ds(d0, _CTILES)])
            return 0

        lax.fori_loop(0, per_w, do_block, 0)

    return select


def kernel(x):
    B, _, D = x.shape
    c = _constants()
    xr, xi, amp = _forward(x, c)
    thr = _make_select(B, D)(amp)
    rec, res = _inverse(x, xr, xi, thr, c)
    return (rec, res)
